# baseline (speedup 1.0000x reference)
import functools

import jax
import jax.numpy as jnp
from jax import lax
from jax.experimental import pallas as pl
from jax.experimental.pallas import tpu as pltpu
from jax.experimental.pallas import tpu_sc as plsc

_GATHER_DNUMS = lax.GatherDimensionNumbers(
    offset_dims=(), collapsed_slice_dims=(0,), start_index_map=(0,))


def _dyn_gather(vec, idx):
    return lax.gather(vec, idx[:, None], dimension_numbers=_GATHER_DNUMS,
                      slice_sizes=(1,),
                      mode=lax.GatherScatterMode.PROMISE_IN_BOUNDS)


NUM_WORKERS = 32
H = 512
W = 512
RB = 32
NCH = H // RB
NTOT = 2 * NCH


def _sc_body(inp_h, tgt_h, ea_h, eb_h, wa_h, wb_h, out_h,
             inb0, inb1, tgb0, tgb1, ea_v, eb_v, wa_v, wb_v, res_v,
             si0, si1, st0, st1):
    c = lax.axis_index("c")
    s = lax.axis_index("s")
    w = s * 2 + c

    pltpu.sync_copy(ea_h, ea_v.at[pl.ds(0, 11)])
    pltpu.sync_copy(eb_h, eb_v.at[pl.ds(0, 11)])
    pltpu.sync_copy(wa_h, wa_v.at[pl.ds(0, 10)])
    pltpu.sync_copy(wb_h, wb_v.at[pl.ds(0, 10)])

    zero16 = jnp.zeros((16,), jnp.int32)
    one16 = jnp.ones((16,), jnp.int32)

    consts = []
    for evref, wvref in ((ea_v, wa_v), (eb_v, wb_v)):
        evec = evref[...]
        wvec = wvref[...]
        e0 = _dyn_gather(evec, zero16)
        e1 = _dyn_gather(evec, one16)
        inv = 1.0 / (e1 - e0)
        nc0 = -e0 * inv
        consts.append((wvec, inv, nc0))

    inbufs = (inb0, inb1)
    tgbufs = (tgb0, tgb1)
    sems_i = (si0, si1)
    sems_t = (st0, st1)

    def start(k, p):
        ch = 0 if k < NCH else 1
        r0 = (k - ch * NCH) * RB
        ci = pltpu.async_copy(inp_h.at[w, ch, pl.ds(r0, RB), :],
                              inbufs[p], sems_i[p])
        ct = pltpu.async_copy(tgt_h.at[w, ch, pl.ds(r0, RB), :],
                              tgbufs[p], sems_t[p])
        return ci, ct

    accs = tuple(jnp.zeros((16,), jnp.float32) for _ in range(4))
    pending = {0: start(0, 0)}
    for k in range(NTOT):
        p = k & 1
        if k + 1 < NTOT:
            pending[1 - p] = start(k + 1, 1 - p)
        ci, ct = pending[p]
        ci.wait()
        ct.wait()
        wvec, inv, nc0 = consts[0] if k < NCH else consts[1]
        inb = inbufs[p]
        tgb = tgbufs[p]

        def step(i, a, inb=inb, tgb=tgb, wvec=wvec, inv=inv, nc0=nc0):
            out = []
            for u in range(4):
                g = i * 4 + u
                r = g >> 5
                col = (g & 31) * 16
                x = tgb[r, pl.ds(col, 16)]
                y = inb[r, pl.ds(col, 16)]
                t = x * inv + nc0
                ii = jnp.minimum(jnp.maximum(t.astype(jnp.int32), 0), 9)
                wv = _dyn_gather(wvec, ii)
                ad = jnp.abs(y - x)
                m = jnp.minimum(ad, 1.0)
                l = m * (0.5 * m) + (ad - m)
                out.append(l * wv + a[u])
            return tuple(out)

        accs = lax.fori_loop(0, RB * (W // 16) // 4, step, accs, unroll=2)

    res_v[...] = (accs[0] + accs[1]) + (accs[2] + accs[3])
    pltpu.sync_copy(res_v, out_h.at[w])


def _finish_body(p_ref, o_ref):
    o_ref[0, 0] = jnp.sum(p_ref[...]) * (1.0 / float(32 * H * W))


def kernel(input, target, bin_edge_a, bin_edge_b, weights_a, weights_b):
    mesh = plsc.VectorSubcoreMesh(core_axis_name="c", subcore_axis_name="s")
    sc = functools.partial(
        pl.kernel,
        mesh=mesh,
        out_type=jax.ShapeDtypeStruct((NUM_WORKERS, 16), jnp.float32),
        compiler_params=pltpu.CompilerParams(use_tc_tiling_on_sc=True),
        scratch_types=[
            pltpu.VMEM((RB, W), jnp.float32),
            pltpu.VMEM((RB, W), jnp.float32),
            pltpu.VMEM((RB, W), jnp.float32),
            pltpu.VMEM((RB, W), jnp.float32),
            pltpu.VMEM((16,), jnp.float32),
            pltpu.VMEM((16,), jnp.float32),
            pltpu.VMEM((16,), jnp.float32),
            pltpu.VMEM((16,), jnp.float32),
            pltpu.VMEM((16,), jnp.float32),
            pltpu.SemaphoreType.DMA,
            pltpu.SemaphoreType.DMA,
            pltpu.SemaphoreType.DMA,
            pltpu.SemaphoreType.DMA,
        ],
    )(_sc_body)
    partials = sc(input, target, bin_edge_a, bin_edge_b, weights_a, weights_b)

    out = pl.pallas_call(
        _finish_body,
        out_shape=jax.ShapeDtypeStruct((1, 1), jnp.float32),
        out_specs=pl.BlockSpec(memory_space=pltpu.SMEM),
    )(partials)
    return out[0, 0]


# speedup vs baseline: 1.0179x; 1.0179x over previous
import functools

import jax
import jax.numpy as jnp
from jax import lax
from jax.experimental import pallas as pl
from jax.experimental.pallas import tpu as pltpu
from jax.experimental.pallas import tpu_sc as plsc

_GATHER_DNUMS = lax.GatherDimensionNumbers(
    offset_dims=(), collapsed_slice_dims=(0,), start_index_map=(0,))


def _dyn_gather(vec, idx):
    return lax.gather(vec, idx[:, None], dimension_numbers=_GATHER_DNUMS,
                      slice_sizes=(1,),
                      mode=lax.GatherScatterMode.PROMISE_IN_BOUNDS)


NUM_WORKERS = 32
H = 512
W = 512
RB = 32
NCH = H // RB
NTOT = 2 * NCH


def _sc_body(inp_h, tgt_h, ea_h, eb_h, wa_h, wb_h, out_h,
             inb0, inb1, tgb0, tgb1, ea_v, eb_v, wa_v, wb_v, res_v,
             si0, si1, st0, st1):
    c = lax.axis_index("c")
    s = lax.axis_index("s")
    w = s * 2 + c

    pltpu.sync_copy(ea_h, ea_v.at[pl.ds(0, 11)])
    pltpu.sync_copy(eb_h, eb_v.at[pl.ds(0, 11)])
    pltpu.sync_copy(wa_h, wa_v.at[pl.ds(0, 10)])
    pltpu.sync_copy(wb_h, wb_v.at[pl.ds(0, 10)])

    zero16 = jnp.zeros((16,), jnp.int32)
    one16 = jnp.ones((16,), jnp.int32)

    consts = []
    for evref, wvref in ((ea_v, wa_v), (eb_v, wb_v)):
        evec = evref[...]
        wvec = wvref[...]
        e0 = _dyn_gather(evec, zero16)
        e1 = _dyn_gather(evec, one16)
        inv = 1.0 / (e1 - e0)
        nc0 = -e0 * inv
        consts.append((wvec, inv, nc0))

    inbufs = (inb0, inb1)
    tgbufs = (tgb0, tgb1)
    sems_i = (si0, si1)
    sems_t = (st0, st1)

    def start(k, p):
        ch = 0 if k < NCH else 1
        r0 = (k - ch * NCH) * RB
        ci = pltpu.async_copy(inp_h.at[w, ch, pl.ds(r0, RB), :],
                              inbufs[p], sems_i[p])
        ct = pltpu.async_copy(tgt_h.at[w, ch, pl.ds(r0, RB), :],
                              tgbufs[p], sems_t[p])
        return ci, ct

    acc = jnp.zeros((16,), jnp.float32)
    pending = {0: start(0, 0)}
    for k in range(NTOT):
        p = k & 1
        if k + 1 < NTOT:
            pending[1 - p] = start(k + 1, 1 - p)
        ci, ct = pending[p]
        ci.wait()
        ct.wait()
        wvec, inv, nc0 = consts[0] if k < NCH else consts[1]
        inb = inbufs[p]
        tgb = tgbufs[p]

        def step(i, a, inb=inb, tgb=tgb, wvec=wvec, inv=inv, nc0=nc0):
            r = i >> 5
            col = (i & 31) * 16
            x = tgb[r, pl.ds(col, 16)]
            y = inb[r, pl.ds(col, 16)]
            t = x * inv + nc0
            ii = jnp.minimum(jnp.maximum(t.astype(jnp.int32), 0), 9)
            wv = _dyn_gather(wvec, ii)
            ad = jnp.abs(y - x)
            m = jnp.minimum(ad, 1.0)
            l = m * (0.5 * m) + (ad - m)
            return l * wv + a

        acc = lax.fori_loop(0, RB * (W // 16), step, acc, unroll=16)

    res_v[...] = acc
    pltpu.sync_copy(res_v, out_h.at[w])


def _finish_body(p_ref, o_ref):
    o_ref[0, 0] = jnp.sum(p_ref[...]) * (1.0 / float(32 * H * W))


def kernel(input, target, bin_edge_a, bin_edge_b, weights_a, weights_b):
    mesh = plsc.VectorSubcoreMesh(core_axis_name="c", subcore_axis_name="s")
    sc = functools.partial(
        pl.kernel,
        mesh=mesh,
        out_type=jax.ShapeDtypeStruct((NUM_WORKERS, 16), jnp.float32),
        compiler_params=pltpu.CompilerParams(use_tc_tiling_on_sc=True),
        scratch_types=[
            pltpu.VMEM((RB, W), jnp.float32),
            pltpu.VMEM((RB, W), jnp.float32),
            pltpu.VMEM((RB, W), jnp.float32),
            pltpu.VMEM((RB, W), jnp.float32),
            pltpu.VMEM((16,), jnp.float32),
            pltpu.VMEM((16,), jnp.float32),
            pltpu.VMEM((16,), jnp.float32),
            pltpu.VMEM((16,), jnp.float32),
            pltpu.VMEM((16,), jnp.float32),
            pltpu.SemaphoreType.DMA,
            pltpu.SemaphoreType.DMA,
            pltpu.SemaphoreType.DMA,
            pltpu.SemaphoreType.DMA,
        ],
    )(_sc_body)
    partials = sc(input, target, bin_edge_a, bin_edge_b, weights_a, weights_b)

    out = pl.pallas_call(
        _finish_body,
        out_shape=jax.ShapeDtypeStruct((1, 1), jnp.float32),
        out_specs=pl.BlockSpec(memory_space=pltpu.SMEM),
    )(partials)
    return out[0, 0]


# speedup vs baseline: 1.0523x; 1.0338x over previous
import functools

import jax
import jax.numpy as jnp
from jax import lax
from jax.experimental import pallas as pl
from jax.experimental.pallas import tpu as pltpu
from jax.experimental.pallas import tpu_sc as plsc

_GATHER_DNUMS = lax.GatherDimensionNumbers(
    offset_dims=(), collapsed_slice_dims=(0,), start_index_map=(0,))


def _dyn_gather(vec, idx):
    return lax.gather(vec, idx[:, None], dimension_numbers=_GATHER_DNUMS,
                      slice_sizes=(1,),
                      mode=lax.GatherScatterMode.PROMISE_IN_BOUNDS)


NUM_WORKERS = 32
H = 512
W = 512
RB = 32
NCH = H // RB
NTOT = 2 * NCH


def _sc_body(inp_h, tgt_h, ea_h, eb_h, wa_h, wb_h, out_h,
             inb0, inb1, tgb0, tgb1, ea_v, eb_v, wa_v, wb_v, res_v,
             si0, si1, st0, st1):
    c = lax.axis_index("c")
    s = lax.axis_index("s")
    w = s * 2 + c

    pltpu.sync_copy(ea_h, ea_v.at[pl.ds(0, 11)])
    pltpu.sync_copy(eb_h, eb_v.at[pl.ds(0, 11)])
    pltpu.sync_copy(wa_h, wa_v.at[pl.ds(0, 10)])
    pltpu.sync_copy(wb_h, wb_v.at[pl.ds(0, 10)])

    zero16 = jnp.zeros((16,), jnp.int32)
    one16 = jnp.ones((16,), jnp.int32)

    consts = []
    for evref, wvref in ((ea_v, wa_v), (eb_v, wb_v)):
        evec = evref[...]
        wvec = wvref[...]
        e0 = _dyn_gather(evec, zero16)
        e1 = _dyn_gather(evec, one16)
        inv = 1.0 / (e1 - e0)
        nc0 = -e0 * inv
        consts.append((wvec, inv, nc0))

    inbufs = (inb0, inb1)
    tgbufs = (tgb0, tgb1)
    sems_i = (si0, si1)
    sems_t = (st0, st1)

    def start(k, p):
        ch = 0 if k < NCH else 1
        r0 = (k - ch * NCH) * RB
        ci = pltpu.async_copy(inp_h.at[w, ch, pl.ds(r0, RB), :],
                              inbufs[p], sems_i[p])
        ct = pltpu.async_copy(tgt_h.at[w, ch, pl.ds(r0, RB), :],
                              tgbufs[p], sems_t[p])
        return ci, ct

    acc = jnp.zeros((16,), jnp.float32)
    pending = {0: start(0, 0)}
    for k in range(NTOT):
        p = k & 1
        if k + 1 < NTOT:
            pending[1 - p] = start(k + 1, 1 - p)
        ci, ct = pending[p]
        ci.wait()
        ct.wait()
        wvec, inv, nc0 = consts[0] if k < NCH else consts[1]
        inb = inbufs[p]
        tgb = tgbufs[p]

        def step(i, a, inb=inb, tgb=tgb, wvec=wvec, inv=inv, nc0=nc0):
            r = i >> 5
            col = (i & 31) * 16
            x = tgb[r, pl.ds(col, 16)]
            y = inb[r, pl.ds(col, 16)]
            t = x * inv + nc0
            ii = jnp.minimum(jnp.maximum(t.astype(jnp.int32), 0), 9)
            wv = _dyn_gather(wvec, ii)
            ad = jnp.abs(y - x)
            m = jnp.minimum(ad, 1.0)
            l = m * (0.5 * m) + (ad - m)
            return l * wv + a

        acc = lax.fori_loop(0, RB * (W // 16), step, acc, unroll=8)

    res_v[...] = acc
    pltpu.sync_copy(res_v, out_h.at[w])


def _finish_body(p_ref, o_ref):
    o_ref[0, 0] = jnp.sum(p_ref[...]) * (1.0 / float(32 * H * W))


def kernel(input, target, bin_edge_a, bin_edge_b, weights_a, weights_b):
    mesh = plsc.VectorSubcoreMesh(core_axis_name="c", subcore_axis_name="s")
    sc = functools.partial(
        pl.kernel,
        mesh=mesh,
        out_type=jax.ShapeDtypeStruct((NUM_WORKERS, 16), jnp.float32),
        compiler_params=pltpu.CompilerParams(use_tc_tiling_on_sc=True),
        scratch_types=[
            pltpu.VMEM((RB, W), jnp.float32),
            pltpu.VMEM((RB, W), jnp.float32),
            pltpu.VMEM((RB, W), jnp.float32),
            pltpu.VMEM((RB, W), jnp.float32),
            pltpu.VMEM((16,), jnp.float32),
            pltpu.VMEM((16,), jnp.float32),
            pltpu.VMEM((16,), jnp.float32),
            pltpu.VMEM((16,), jnp.float32),
            pltpu.VMEM((16,), jnp.float32),
            pltpu.SemaphoreType.DMA,
            pltpu.SemaphoreType.DMA,
            pltpu.SemaphoreType.DMA,
            pltpu.SemaphoreType.DMA,
        ],
    )(_sc_body)
    partials = sc(input, target, bin_edge_a, bin_edge_b, weights_a, weights_b)

    out = pl.pallas_call(
        _finish_body,
        out_shape=jax.ShapeDtypeStruct((1, 1), jnp.float32),
        out_specs=pl.BlockSpec(memory_space=pltpu.SMEM),
    )(partials)
    return out[0, 0]


# speedup vs baseline: 1.0731x; 1.0198x over previous
import functools

import jax
import jax.numpy as jnp
from jax import lax
from jax.experimental import pallas as pl
from jax.experimental.pallas import tpu as pltpu
from jax.experimental.pallas import tpu_sc as plsc

_GATHER_DNUMS = lax.GatherDimensionNumbers(
    offset_dims=(), collapsed_slice_dims=(0,), start_index_map=(0,))


def _dyn_gather(vec, idx):
    return lax.gather(vec, idx[:, None], dimension_numbers=_GATHER_DNUMS,
                      slice_sizes=(1,),
                      mode=lax.GatherScatterMode.PROMISE_IN_BOUNDS)


NUM_WORKERS = 32
H = 512
W = 512
RB = 32
NCH = H // RB
NTOT = 2 * NCH


def _sc_body(inp_h, tgt_h, ea_h, eb_h, wa_h, wb_h, out_h,
             inb0, inb1, tgb0, tgb1, ea_v, eb_v, wa_v, wb_v, res_v,
             si0, si1, st0, st1):
    c = lax.axis_index("c")
    s = lax.axis_index("s")
    w = s * 2 + c

    zero16 = jnp.zeros((16,), jnp.int32)
    one16 = jnp.ones((16,), jnp.int32)

    inbufs = (inb0, inb1)
    tgbufs = (tgb0, tgb1)
    sems_i = (si0, si1)
    sems_t = (st0, st1)

    def start(k, p):
        ch = 0 if k < NCH else 1
        r0 = (k - ch * NCH) * RB
        ci = pltpu.async_copy(inp_h.at[w, ch, pl.ds(r0, RB), :],
                              inbufs[p], sems_i[p])
        ct = pltpu.async_copy(tgt_h.at[w, ch, pl.ds(r0, RB), :],
                              tgbufs[p], sems_t[p])
        return ci, ct

    pending = {0: start(0, 0)}
    pltpu.sync_copy(ea_h, ea_v.at[pl.ds(0, 11)])
    pltpu.sync_copy(eb_h, eb_v.at[pl.ds(0, 11)])
    pltpu.sync_copy(wa_h, wa_v.at[pl.ds(0, 10)])
    pltpu.sync_copy(wb_h, wb_v.at[pl.ds(0, 10)])

    consts = []
    for evref, wvref in ((ea_v, wa_v), (eb_v, wb_v)):
        evec = evref[...]
        wvec = wvref[...]
        e0 = _dyn_gather(evec, zero16)
        e1 = _dyn_gather(evec, one16)
        inv = 1.0 / (e1 - e0)
        nc0 = -e0 * inv
        consts.append((wvec, inv, nc0))

    acc = jnp.zeros((16,), jnp.float32)
    for k in range(NTOT):
        p = k & 1
        if k + 1 < NTOT:
            pending[1 - p] = start(k + 1, 1 - p)
        ci, ct = pending[p]
        ci.wait()
        ct.wait()
        wvec, inv, nc0 = consts[0] if k < NCH else consts[1]
        inb = inbufs[p]
        tgb = tgbufs[p]

        def step(i, a, inb=inb, tgb=tgb, wvec=wvec, inv=inv, nc0=nc0):
            r = i >> 5
            col = (i & 31) * 16
            x = tgb[r, pl.ds(col, 16)]
            y = inb[r, pl.ds(col, 16)]
            t = x * inv + nc0
            ii = jnp.minimum(jnp.maximum(t.astype(jnp.int32), 0), 9)
            wv = _dyn_gather(wvec, ii)
            ad = jnp.abs(y - x)
            m = jnp.minimum(ad, 1.0)
            l = m * (0.5 * m) + (ad - m)
            return l * wv + a

        acc = lax.fori_loop(0, RB * (W // 16), step, acc, unroll=8)

    res_v[...] = acc
    pltpu.sync_copy(res_v, out_h.at[w])


def _finish_body(p_ref, o_ref):
    o_ref[0, 0] = jnp.sum(p_ref[...]) * (1.0 / float(32 * H * W))


def kernel(input, target, bin_edge_a, bin_edge_b, weights_a, weights_b):
    mesh = plsc.VectorSubcoreMesh(core_axis_name="c", subcore_axis_name="s")
    sc = functools.partial(
        pl.kernel,
        mesh=mesh,
        out_type=jax.ShapeDtypeStruct((NUM_WORKERS, 16), jnp.float32),
        compiler_params=pltpu.CompilerParams(use_tc_tiling_on_sc=True),
        scratch_types=[
            pltpu.VMEM((RB, W), jnp.float32),
            pltpu.VMEM((RB, W), jnp.float32),
            pltpu.VMEM((RB, W), jnp.float32),
            pltpu.VMEM((RB, W), jnp.float32),
            pltpu.VMEM((16,), jnp.float32),
            pltpu.VMEM((16,), jnp.float32),
            pltpu.VMEM((16,), jnp.float32),
            pltpu.VMEM((16,), jnp.float32),
            pltpu.VMEM((16,), jnp.float32),
            pltpu.SemaphoreType.DMA,
            pltpu.SemaphoreType.DMA,
            pltpu.SemaphoreType.DMA,
            pltpu.SemaphoreType.DMA,
        ],
    )(_sc_body)
    partials = sc(input, target, bin_edge_a, bin_edge_b, weights_a, weights_b)

    out = pl.pallas_call(
        _finish_body,
        out_shape=jax.ShapeDtypeStruct((1, 1), jnp.float32),
        out_specs=pl.BlockSpec(memory_space=pltpu.SMEM),
    )(partials)
    return out[0, 0]
